# Initial kernel scaffold; baseline (speedup 1.0000x reference)
import jax
import jax.numpy as jnp
from jax.experimental import pallas as pl


def kernel(x, edge_index, W1, b1, W2, b2, W_skip):
    raise NotImplementedError("write your pallas kernel here")


# speedup vs baseline: 10.7860x; 10.7860x over previous
import functools

import jax
import jax.numpy as jnp
from jax import lax
from jax.experimental import pallas as pl
from jax.experimental.pallas import tpu as pltpu
from jax.experimental.pallas import tpu_sc as plsc

NC = 2
NS = 16
NW = NC * NS
CH = 128


def _sc_mesh():
    return plsc.VectorSubcoreMesh(
        core_axis_name="c", subcore_axis_name="s",
        num_cores=NC, num_subcores=NS)


def _deg_kernel(np_, nch):
    rows_per_tile = np_ // NS

    @functools.partial(
        pl.kernel,
        out_type=jax.ShapeDtypeStruct((NC, np_, 128), jnp.float32),
        mesh=_sc_mesh(),
        scratch_types=[
            pltpu.VMEM((nch, CH), jnp.int32),
            pltpu.VMEM((CH, 128), jnp.float32),
            pltpu.VMEM_SHARED((np_, 128), jnp.float32),
        ],
    )
    def deg_kernel(dst_hbm, ones_hbm, zeros_hbm, out_hbm, dst_v, ones_v, acc_sh):
        cid = lax.axis_index("c")
        sid = lax.axis_index("s")
        wid = sid * NC + cid
        base = sid * rows_per_tile
        pltpu.sync_copy(zeros_hbm.at[pl.ds(base, rows_per_tile)],
                        acc_sh.at[pl.ds(base, rows_per_tile)])
        pltpu.sync_copy(dst_hbm.at[wid], dst_v)
        pltpu.sync_copy(ones_hbm, ones_v)
        plsc.subcore_barrier()

        def body(j, carry):
            pltpu.sync_copy(ones_v, acc_sh.at[dst_v.at[j]], add=True)
            return carry
        lax.fori_loop(0, nch, body, 0)
        plsc.subcore_barrier()
        pltpu.sync_copy(acc_sh.at[pl.ds(base, rows_per_tile)],
                        out_hbm.at[cid, pl.ds(base, rows_per_tile)])

    return deg_kernel


def _agg_kernel(np_, nch, w):
    rows_per_tile = np_ // NS

    @functools.partial(
        pl.kernel,
        out_type=jax.ShapeDtypeStruct((NC, np_, w), jnp.float32),
        mesh=_sc_mesh(),
        scratch_types=[
            pltpu.VMEM((nch, CH), jnp.int32),
            pltpu.VMEM((nch, CH), jnp.int32),
            pltpu.VMEM((CH, w), jnp.float32),
            pltpu.VMEM_SHARED((np_, w), jnp.float32),
        ],
    )
    def agg_kernel(hs_hbm, src_hbm, dst_hbm, zeros_hbm, out_hbm,
                   src_v, dst_v, buf, acc_sh):
        cid = lax.axis_index("c")
        sid = lax.axis_index("s")
        wid = sid * NC + cid
        base = sid * rows_per_tile
        pltpu.sync_copy(zeros_hbm.at[pl.ds(base, rows_per_tile)],
                        acc_sh.at[pl.ds(base, rows_per_tile)])
        pltpu.sync_copy(src_hbm.at[wid], src_v)
        pltpu.sync_copy(dst_hbm.at[wid], dst_v)
        plsc.subcore_barrier()

        def body(j, carry):
            pltpu.sync_copy(hs_hbm.at[src_v.at[j]], buf)
            pltpu.sync_copy(buf, acc_sh.at[dst_v.at[j]], add=True)
            return carry
        lax.fori_loop(0, nch, body, 0)
        plsc.subcore_barrier()
        pltpu.sync_copy(acc_sh.at[pl.ds(base, rows_per_tile)],
                        out_hbm.at[cid, pl.ds(base, rows_per_tile)])

    return agg_kernel


def _mm1_body(x_ref, w1_ref, w2a_ref, wsk_ref, b2_ref,
              h1_ref, p2a_ref, skp_ref):
    xb = x_ref[...]
    h1_ref[...] = jnp.dot(xb, w1_ref[...], preferred_element_type=jnp.float32)
    p2a_ref[...] = jnp.dot(xb, w2a_ref[...], preferred_element_type=jnp.float32)
    skp_ref[...] = (jnp.dot(xb, wsk_ref[...], preferred_element_type=jnp.float32)
                    + b2_ref[...])


def _disb_body(dp_ref, out_ref):
    out_ref[...] = lax.rsqrt(dp_ref[0] + dp_ref[1] + 1.0)


def _scale_body(disb_ref, h1_ref, b1_ref, hs1_ref, sb1_ref):
    dis = disb_ref[...]
    h1 = h1_ref[...]
    hs1_ref[...] = h1 * dis
    sb1_ref[...] = h1 * (dis * dis) + b1_ref[...]


def _mid_body(disb_ref, a1_ref, sb1_ref, p2a_ref, skp_ref, w2b_ref,
              hs2_ref, base_ref):
    dis = disb_ref[...]
    x1 = jnp.maximum(dis * (a1_ref[0] + a1_ref[1]) + sb1_ref[...], 0.0)
    h2 = p2a_ref[...] + jnp.dot(x1, w2b_ref[...],
                                preferred_element_type=jnp.float32)
    hs2_ref[...] = h2 * dis
    base_ref[...] = h2 * (dis * dis) + skp_ref[...]


def _fin_body(disb_ref, a2_ref, base_ref, out_ref):
    dis = disb_ref[...]
    out_ref[...] = dis * (a2_ref[0] + a2_ref[1]) + base_ref[...]


def kernel(x, edge_index, W1, b1, W2, b2, W_skip):
    n, d = x.shape
    hdim = W1.shape[1]
    ncls = W_skip.shape[1]
    e = edge_index.shape[1]

    np_ = ((n + 1 + 127) // 128) * 128
    nch = (e + NW * CH - 1) // (NW * CH)
    ep = NW * nch * CH
    c2 = ((ncls + 127) // 128) * 128

    ei = edge_index.astype(jnp.int32)
    pad = jnp.full((ep - e,), n, jnp.int32)
    src3 = jnp.concatenate([ei[0], pad]).reshape(NW, nch, CH)
    dst3 = jnp.concatenate([ei[1], pad]).reshape(NW, nch, CH)

    x_pad = jnp.pad(x, ((0, np_ - n), (0, 0)))
    b1r = b1.reshape(1, hdim)
    b2r = jnp.pad(b2, (0, c2 - ncls)).reshape(1, c2)
    W2a = jnp.pad(W2[:d], ((0, 0), (0, c2 - ncls)))
    W2b = jnp.pad(W2[d:], ((0, 0), (0, c2 - ncls)))
    Wsk = jnp.pad(W_skip, ((0, 0), (0, c2 - ncls)))
    zh = jnp.zeros((np_, hdim), jnp.float32)
    zc = jnp.zeros((np_, c2), jnp.float32)

    f32 = jnp.float32
    BR = np_ // 8
    grid = (np_ // BR,)

    def full(shape):
        return pl.BlockSpec(shape, lambda i: tuple(0 for _ in shape))

    rows = lambda w: pl.BlockSpec((BR, w), lambda i: (i, 0))
    parts = lambda w: pl.BlockSpec((NC, BR, w), lambda i: (0, i, 0))

    ones128 = jnp.ones((CH, 128), jnp.float32)
    dp = _deg_kernel(np_, nch)(dst3, ones128, zh)

    disb = pl.pallas_call(
        _disb_body,
        grid=grid,
        in_specs=[parts(128)],
        out_specs=rows(128),
        out_shape=jax.ShapeDtypeStruct((np_, 128), f32),
    )(dp)

    h1, p2a, skp = pl.pallas_call(
        _mm1_body,
        grid=grid,
        in_specs=[rows(d), full((d, hdim)), full((d, c2)), full((d, c2)),
                  full((1, c2))],
        out_specs=[rows(hdim), rows(c2), rows(c2)],
        out_shape=[jax.ShapeDtypeStruct((np_, hdim), f32),
                   jax.ShapeDtypeStruct((np_, c2), f32),
                   jax.ShapeDtypeStruct((np_, c2), f32)],
    )(x_pad, W1, W2a, Wsk, b2r)

    hs1, sb1 = pl.pallas_call(
        _scale_body,
        grid=grid,
        in_specs=[rows(128), rows(hdim), full((1, hdim))],
        out_specs=[rows(hdim), rows(hdim)],
        out_shape=[jax.ShapeDtypeStruct((np_, hdim), f32),
                   jax.ShapeDtypeStruct((np_, hdim), f32)],
    )(disb, h1, b1r)

    a1 = _agg_kernel(np_, nch, hdim)(hs1, src3, dst3, zh)

    hs2, base = pl.pallas_call(
        _mid_body,
        grid=grid,
        in_specs=[rows(128), parts(hdim), rows(hdim), rows(c2), rows(c2),
                  full((hdim, c2))],
        out_specs=[rows(c2), rows(c2)],
        out_shape=[jax.ShapeDtypeStruct((np_, c2), f32),
                   jax.ShapeDtypeStruct((np_, c2), f32)],
    )(disb, a1, sb1, p2a, skp, W2b)

    a2 = _agg_kernel(np_, nch, c2)(hs2, src3, dst3, zc)

    out = pl.pallas_call(
        _fin_body,
        grid=grid,
        in_specs=[rows(128), parts(c2), rows(c2)],
        out_specs=rows(c2),
        out_shape=jax.ShapeDtypeStruct((np_, c2), f32),
    )(disb, a2, base)

    return out[:n, :ncls]
